# Initial kernel scaffold; baseline (speedup 1.0000x reference)
import functools

import jax
import jax.numpy as jnp
from jax import lax
from jax.experimental import pallas as pl
from jax.experimental.pallas import tpu as pltpu
from jax.experimental.pallas import tpu_sc as plsc

_VOCAB = 1000000
_DIM = 64
_B = 16384
_K = 20

_NW = 32
_NB = _B // _NW
_BLK = 32
_NBLK = _NB // _BLK
_NEG_BLK = _BLK * _K
_ICH = 128
_NCH = _NEG_BLK // _ICH


def _sc_body(tgt_hbm, pos_hbm, neg_hbm, tw_hbm, cw_hbm,
             pos_out, neg_out,
             tidx, pidx, nidx, t_v, p_v, n_v, ps_v, ns_v, sem):
    wid = lax.axis_index("s") * 2 + lax.axis_index("c")

    pltpu.sync_copy(tgt_hbm.at[wid], tidx)
    pltpu.sync_copy(pos_hbm.at[wid], pidx)
    pltpu.sync_copy(neg_hbm.at[wid], nidx)

    def blk_body(blk, carry):
        cps = [
            pltpu.async_copy(tw_hbm.at[tidx.at[blk]], t_v, sem),
            pltpu.async_copy(cw_hbm.at[pidx.at[blk]], p_v, sem),
        ]
        for j in range(_NCH):
            cps.append(pltpu.async_copy(
                cw_hbm.at[nidx.at[blk * _NCH + j]],
                n_v.at[pl.ds(j * _ICH, _ICH)], sem))
        for cp in cps:
            cp.wait()

        def tgt_body(i, carry2):
            t0 = t_v[i, pl.ds(0, 16)]
            t1 = t_v[i, pl.ds(16, 16)]
            t2 = t_v[i, pl.ds(32, 16)]
            t3 = t_v[i, pl.ds(48, 16)]
            p0 = p_v[i, pl.ds(0, 16)]
            p1 = p_v[i, pl.ds(16, 16)]
            p2 = p_v[i, pl.ds(32, 16)]
            p3 = p_v[i, pl.ds(48, 16)]
            b = blk * _BLK + i
            ps_v[b] = jnp.sum(t0 * p0 + t1 * p1 + t2 * p2 + t3 * p3)
            for k in range(_K):
                r = i * _K + k
                n0 = n_v[r, pl.ds(0, 16)]
                n1 = n_v[r, pl.ds(16, 16)]
                n2 = n_v[r, pl.ds(32, 16)]
                n3 = n_v[r, pl.ds(48, 16)]
                ns_v[b * _K + k] = jnp.sum(n0 * t0 + n1 * t1 + n2 * t2 + n3 * t3)
            return carry2

        lax.fori_loop(0, _BLK, tgt_body, 0)
        return carry

    lax.fori_loop(0, _NBLK, blk_body, 0)

    pltpu.sync_copy(ps_v, pos_out.at[wid])
    pltpu.sync_copy(ns_v, neg_out.at[wid])


_sc_scores = functools.partial(
    pl.kernel,
    out_type=(
        jax.ShapeDtypeStruct((_NW, _NB), jnp.float32),
        jax.ShapeDtypeStruct((_NW, _NB * _K), jnp.float32),
    ),
    mesh=plsc.VectorSubcoreMesh(core_axis_name="c", subcore_axis_name="s"),
    scratch_types=[
        pltpu.VMEM((_NBLK, _BLK), jnp.int32),
        pltpu.VMEM((_NBLK, _BLK), jnp.int32),
        pltpu.VMEM((_NBLK * _NCH, _ICH), jnp.int32),
        pltpu.VMEM((_BLK, _DIM), jnp.float32),
        pltpu.VMEM((_BLK, _DIM), jnp.float32),
        pltpu.VMEM((_NEG_BLK, _DIM), jnp.float32),
        pltpu.VMEM((_NB,), jnp.float32),
        pltpu.VMEM((_NB * _K,), jnp.float32),
        pltpu.SemaphoreType.DMA,
    ],
)(_sc_body)


def _tc_loss_body(ps_ref, ns_ref, o_ref):
    def logsig(x):
        return jnp.minimum(x, 0.0) - jnp.log1p(jnp.exp(-jnp.abs(x)))

    ps = ps_ref[...]
    ns = ns_ref[...]
    pos_loss = -jnp.sum(jnp.sum(logsig(ps), axis=0)) / _B
    neg_loss = -jnp.sum(jnp.sum(logsig(-ns), axis=0)) / (_B * _K)
    o_ref[0, 0] = pos_loss + neg_loss


_tc_loss = pl.pallas_call(
    _tc_loss_body,
    out_shape=jax.ShapeDtypeStruct((1, 1), jnp.float32),
    out_specs=pl.BlockSpec(memory_space=pltpu.SMEM),
)


def kernel(target_ids, pos_ids, neg_ids, target_w, context_w):
    tgt = target_ids.astype(jnp.int32).reshape(_NW, _NBLK, _BLK)
    pos = pos_ids.astype(jnp.int32).reshape(_NW, _NBLK, _BLK)
    neg = neg_ids.astype(jnp.int32).reshape(_NW, _NBLK * _NCH, _ICH)
    ps, ns = _sc_scores(tgt, pos, neg, target_w, context_w)
    loss = _tc_loss(ps.reshape(128, 128), ns.reshape(2560, 128))
    return loss[0, 0]


# speedup vs baseline: 5.3021x; 5.3021x over previous
import functools

import jax
import jax.numpy as jnp
from jax import lax
from jax.experimental import pallas as pl
from jax.experimental.pallas import tpu as pltpu
from jax.experimental.pallas import tpu_sc as plsc

_VOCAB = 1000000
_DIM = 64
_B = 16384
_K = 20

_NW = 32
_NB = _B // _NW
_BLK = 32
_NBLK = _NB // _BLK
_NEG_BLK = _BLK * _K
_ICH = 128
_NCH = _NEG_BLK // _ICH


def _sc_body(tgt_hbm, pos_hbm, neg_hbm, tw_hbm, cw_hbm,
             pos_out, neg_out,
             tidx, pidx, nidx, t_v, p_v, n_v, ps_v, ns_v, sem):
    wid = lax.axis_index("s") * 2 + lax.axis_index("c")

    pltpu.sync_copy(tgt_hbm.at[wid], tidx)
    pltpu.sync_copy(pos_hbm.at[wid], pidx)
    pltpu.sync_copy(neg_hbm.at[wid], nidx)

    iota16 = lax.iota(jnp.int32, 16)

    def load_row(ref, r):
        return [ref[r, pl.ds(c, 16)] for c in (0, 16, 32, 48)]

    def blk_body(blk, carry):
        cps = [
            pltpu.async_copy(tw_hbm.at[tidx.at[blk]], t_v, sem),
            pltpu.async_copy(cw_hbm.at[pidx.at[blk]], p_v, sem),
        ]
        for j in range(_NCH):
            cps.append(pltpu.async_copy(
                cw_hbm.at[nidx.at[blk * _NCH + j]],
                n_v.at[pl.ds(j * _ICH, _ICH)], sem))
        for cp in cps:
            cp.wait()

        def pos_grp(h, c2):
            acc = jnp.zeros((16,), jnp.float32)
            for l in range(16):
                i = h * 16 + l
                t = load_row(t_v, i)
                p = load_row(p_v, i)
                s = jnp.sum(t[0] * p[0] + t[1] * p[1] + t[2] * p[2] + t[3] * p[3])
                acc = jnp.where(iota16 == l, s, acc)
            ps_v[pl.ds(blk * _BLK + h * 16, 16)] = acc
            return c2

        lax.fori_loop(0, 2, pos_grp, 0)

        def neg_grp(g, c2):
            i0 = g * 4
            ts = [load_row(t_v, i0 + j) for j in range(4)]
            base_row = i0 * _K
            out_base = blk * _NEG_BLK + g * 4 * _K
            for v in range(5):
                acc = jnp.zeros((16,), jnp.float32)
                for l in range(16):
                    d = v * 16 + l
                    t = ts[d // _K]
                    n = load_row(n_v, base_row + d)
                    s = jnp.sum(n[0] * t[0] + n[1] * t[1]
                                + n[2] * t[2] + n[3] * t[3])
                    acc = jnp.where(iota16 == l, s, acc)
                ns_v[pl.ds(out_base + v * 16, 16)] = acc
            return c2

        lax.fori_loop(0, 8, neg_grp, 0)
        return carry

    lax.fori_loop(0, _NBLK, blk_body, 0)

    pltpu.sync_copy(ps_v, pos_out.at[wid])
    pltpu.sync_copy(ns_v, neg_out.at[wid])


_sc_scores = functools.partial(
    pl.kernel,
    out_type=(
        jax.ShapeDtypeStruct((_NW, _NB), jnp.float32),
        jax.ShapeDtypeStruct((_NW, _NB * _K), jnp.float32),
    ),
    mesh=plsc.VectorSubcoreMesh(core_axis_name="c", subcore_axis_name="s"),
    compiler_params=pltpu.CompilerParams(
        needs_layout_passes=False, use_tc_tiling_on_sc=False),
    scratch_types=[
        pltpu.VMEM((_NBLK, _BLK), jnp.int32),
        pltpu.VMEM((_NBLK, _BLK), jnp.int32),
        pltpu.VMEM((_NBLK * _NCH, _ICH), jnp.int32),
        pltpu.VMEM((_BLK, _DIM), jnp.float32),
        pltpu.VMEM((_BLK, _DIM), jnp.float32),
        pltpu.VMEM((_NEG_BLK, _DIM), jnp.float32),
        pltpu.VMEM((_NB,), jnp.float32),
        pltpu.VMEM((_NB * _K,), jnp.float32),
        pltpu.SemaphoreType.DMA,
    ],
)(_sc_body)


def _tc_loss_body(ps_ref, ns_ref, o_ref):
    def logsig(x):
        return jnp.minimum(x, 0.0) - jnp.log1p(jnp.exp(-jnp.abs(x)))

    ps = ps_ref[...]
    ns = ns_ref[...]
    pos_loss = -jnp.sum(jnp.sum(logsig(ps), axis=0)) / _B
    neg_loss = -jnp.sum(jnp.sum(logsig(-ns), axis=0)) / (_B * _K)
    o_ref[0, 0] = pos_loss + neg_loss


_tc_loss = pl.pallas_call(
    _tc_loss_body,
    out_shape=jax.ShapeDtypeStruct((1, 1), jnp.float32),
    out_specs=pl.BlockSpec(memory_space=pltpu.SMEM),
)


def kernel(target_ids, pos_ids, neg_ids, target_w, context_w):
    tgt = target_ids.astype(jnp.int32).reshape(_NW, _NBLK, _BLK)
    pos = pos_ids.astype(jnp.int32).reshape(_NW, _NBLK, _BLK)
    neg = neg_ids.astype(jnp.int32).reshape(_NW, _NBLK * _NCH, _ICH)
    ps, ns = _sc_scores(tgt, pos, neg, target_w, context_w)
    loss = _tc_loss(ps.reshape(128, 128), ns.reshape(2560, 128))
    return loss[0, 0]
